# Initial kernel scaffold; baseline (speedup 1.0000x reference)
import jax
import jax.numpy as jnp
from jax.experimental import pallas as pl


def kernel(coords, embeddings):
    raise NotImplementedError("write your pallas kernel here")


# speedup vs baseline: 747.5541x; 747.5541x over previous
import functools

import jax
import jax.numpy as jnp
import numpy as np
from jax import lax
from jax.experimental import pallas as pl
from jax.experimental.pallas import tpu as pltpu
from jax.experimental.pallas import tpu_sc as plsc

PI_2 = int(np.uint32(2654435761).view(np.int32))
PI_3 = 805459861

MAX_ENTRIES = 2**14
NUM_LEVELS = 16
DIM = 2
MIN_RES = 16
MAX_RES = 512
N_POINTS = 1048576

CH = 4096
HALF = N_POINTS // 2
N_CHUNKS = HALF // CH
GROUPS = CH // 16


def _level_meta():
    b = np.exp((np.log(MAX_RES) - np.log(MIN_RES)) / (NUM_LEVELS - 1))
    counts, resolutions = [], []
    for l in range(NUM_LEVELS):
        res = int(np.floor(MIN_RES * (b**l)))
        counts.append(int(min((res + 1) ** 3, MAX_ENTRIES)))
        resolutions.append(res)
    offsets = np.concatenate([[0], np.cumsum(counts)]).astype(np.int64)
    return counts, resolutions, offsets


_COUNTS, _RES, _OFFSETS = _level_meta()


def _sc_body(coords_hbm, table_hbm, resf_hbm, m1_hbm, out_hbm,
             cbuf, t0, t1, o0, o1, resv, m1v):
    l = lax.axis_index("s")
    h = lax.axis_index("c")

    pltpu.sync_copy(resf_hbm, resv)
    pltpu.sync_copy(m1_hbm, m1v)
    pltpu.sync_copy(table_hbm.at[l, 0], t0)
    pltpu.sync_copy(table_hbm.at[l, 1], t1)

    lvec = jnp.full((16,), l, dtype=jnp.int32)
    res_f = plsc.load_gather(resv, [lvec])
    m1 = plsc.load_gather(m1v, [lvec])
    m2 = m1 * m1

    iota3 = lax.iota(jnp.int32, 16) * 3
    half_base = h * HALF

    def make_inner(dense):
        def group_body(g, _):
            ix = iota3 + g * 48
            x = plsc.load_gather(cbuf, [ix])
            y = plsc.load_gather(cbuf, [ix + 1])
            z = plsc.load_gather(cbuf, [ix + 2])
            sx = x * res_f
            sy = y * res_f
            sz = z * res_f
            px = sx.astype(jnp.int32)
            py = sy.astype(jnp.int32)
            pz = sz.astype(jnp.int32)
            fx = sx - px.astype(jnp.float32)
            fy = sy - py.astype(jnp.float32)
            fz = sz - pz.astype(jnp.float32)
            gx = 1.0 - fx
            gy = 1.0 - fy
            gz = 1.0 - fz
            w00 = gy * gz
            w01 = gy * fz
            w10 = fy * gz
            w11 = fy * fz
            if dense:
                y0 = py * m1
                y1 = y0 + m1
                z0 = pz * m2
                z1 = z0 + m2
            else:
                y0 = py * PI_2
                y1 = y0 + PI_2
                z0 = pz * PI_3
                z1 = z0 + PI_3
            if dense:
                b00 = y0 + z0
                b01 = y0 + z1
                b10 = y1 + z0
                b11 = y1 + z1
            else:
                b00 = (y0 ^ z0) & 16383
                b01 = (y0 ^ z1) & 16383
                b10 = (y1 ^ z0) & 16383
                b11 = (y1 ^ z1) & 16383
            px1 = px + 1
            acc0 = jnp.zeros((16,), jnp.float32)
            acc1 = jnp.zeros((16,), jnp.float32)
            for cx, wx in ((px, gx), (px1, fx)):
                for bc, wyz in ((b00, w00), (b01, w01), (b10, w10), (b11, w11)):
                    idx = (cx + bc) if dense else (cx ^ bc)
                    w = wx * wyz
                    acc0 = acc0 + w * plsc.load_gather(t0, [idx])
                    acc1 = acc1 + w * plsc.load_gather(t1, [idx])
            o0[pl.ds(g * 16, 16)] = acc0
            o1[pl.ds(g * 16, 16)] = acc1
            return 0

        lax.fori_loop(0, GROUPS, group_body, 0)

    def chunk_body(i, _):
        base = half_base + i * CH
        pltpu.sync_copy(coords_hbm.at[pl.ds(base * 3, CH * 3)], cbuf)

        @pl.when(l < 2)
        def _():
            make_inner(True)

        @pl.when(l >= 2)
        def _():
            make_inner(False)

        pltpu.sync_copy(o0, out_hbm.at[l, 0, pl.ds(base, CH)])
        pltpu.sync_copy(o1, out_hbm.at[l, 1, pl.ds(base, CH)])
        return 0

    lax.fori_loop(0, N_CHUNKS, chunk_body, 0)


@jax.jit
def _hash_encode_sc(coords_flat, table, resf, m1):
    mesh = plsc.VectorSubcoreMesh(
        core_axis_name="c", subcore_axis_name="s", num_cores=2, num_subcores=16
    )
    f = functools.partial(
        pl.kernel,
        out_type=jax.ShapeDtypeStruct((NUM_LEVELS, DIM, N_POINTS), jnp.float32),
        mesh=mesh,
        compiler_params=pltpu.CompilerParams(needs_layout_passes=False),
        scratch_types=[
            pltpu.VMEM((CH * 3,), jnp.float32),
            pltpu.VMEM((MAX_ENTRIES,), jnp.float32),
            pltpu.VMEM((MAX_ENTRIES,), jnp.float32),
            pltpu.VMEM((CH,), jnp.float32),
            pltpu.VMEM((CH,), jnp.float32),
            pltpu.VMEM((NUM_LEVELS,), jnp.float32),
            pltpu.VMEM((NUM_LEVELS,), jnp.int32),
        ],
    )(_sc_body)
    return f(coords_flat, table, resf, m1)


def kernel(coords, embeddings):
    planes = []
    for l in range(NUM_LEVELS):
        off, cnt = int(_OFFSETS[l]), _COUNTS[l]
        sl = embeddings[off:off + cnt].T
        planes.append(jnp.pad(sl, ((0, 0), (0, MAX_ENTRIES - cnt))))
    table = jnp.stack(planes)
    resf = jnp.asarray(_RES, dtype=jnp.float32)
    m1 = jnp.asarray([r + 1 for r in _RES], dtype=jnp.int32)

    out = _hash_encode_sc(coords.reshape(-1), table, resf, m1)
    return out.transpose(2, 0, 1).reshape(N_POINTS, NUM_LEVELS * DIM)
